# Initial kernel scaffold; baseline (speedup 1.0000x reference)
import jax
import jax.numpy as jnp
from jax.experimental import pallas as pl


def kernel(points, features, neighbors, kernel_points, W_kp, W_unary, b_unary, W_mlp, b_mlp, W_center, b_center, W_var, b_var, W_soft, b_soft):
    raise NotImplementedError("write your pallas kernel here")


# speedup vs baseline: 1.8816x; 1.8816x over previous
import jax
import jax.numpy as jnp
from jax import lax
from jax.experimental import pallas as pl
from jax.experimental.pallas import tpu as pltpu
from jax.experimental.pallas import tpu_sc as plsc

N = 10000
KN = 32
KP = 15
IN_DIM = 128
OUT_DIM = 128
FFD = 128
FREE_DIM = 4
C = 19
KP_EXTENT = 1.2

SC_NC = 2
SC_NS = 16
NW = SC_NC * SC_NS
E = N * KN
PER_W = E // NW
GCHUNK = 400
GITERS = PER_W // GCHUNK


def _sc_gather_body(nb_hbm, feat_hbm, coords_hbm,
                    gf_hbm, gx_hbm, gy_hbm, gz_hbm,
                    coords_v, idx_v, rows_v, gx_v, gy_v, gz_v, sem):
    c = lax.axis_index("c")
    s = lax.axis_index("s")
    wid = s * SC_NC + c
    base = pl.multiple_of(wid * PER_W, 8)
    pltpu.sync_copy(coords_hbm, coords_v)

    def body(j, carry):
        off = pl.multiple_of(base + j * GCHUNK, 8)
        pltpu.sync_copy(nb_hbm.at[pl.ds(off, GCHUNK)], idx_v)
        cp = pltpu.async_copy(feat_hbm.at[idx_v], rows_v, sem)
        for g in range(GCHUNK // 16):
            i16 = idx_v[pl.ds(g * 16, 16)]
            b4 = i16 * 4
            gx_v[pl.ds(g * 16, 16)] = plsc.load_gather(coords_v, [b4])
            gy_v[pl.ds(g * 16, 16)] = plsc.load_gather(coords_v, [b4 + 1])
            gz_v[pl.ds(g * 16, 16)] = plsc.load_gather(coords_v, [b4 + 2])
        cp.wait()
        pltpu.sync_copy(rows_v, gf_hbm.at[pl.ds(off, GCHUNK)])
        pltpu.sync_copy(gx_v, gx_hbm.at[pl.ds(off, GCHUNK)])
        pltpu.sync_copy(gy_v, gy_hbm.at[pl.ds(off, GCHUNK)])
        pltpu.sync_copy(gz_v, gz_hbm.at[pl.ds(off, GCHUNK)])
        return carry

    lax.fori_loop(0, GITERS, body, 0)


@jax.jit
def _sc_gather(nb_flat, features, coords_flat):
    mesh = plsc.VectorSubcoreMesh(core_axis_name="c", subcore_axis_name="s")
    return pl.kernel(
        _sc_gather_body,
        out_type=[
            jax.ShapeDtypeStruct((E, IN_DIM), jnp.float32),
            jax.ShapeDtypeStruct((E,), jnp.float32),
            jax.ShapeDtypeStruct((E,), jnp.float32),
            jax.ShapeDtypeStruct((E,), jnp.float32),
        ],
        mesh=mesh,
        compiler_params=pltpu.CompilerParams(needs_layout_passes=False),
        scratch_types=[
            pltpu.VMEM((4 * N,), jnp.float32),
            pltpu.VMEM((GCHUNK,), jnp.int32),
            pltpu.VMEM((GCHUNK, IN_DIM), jnp.float32),
            pltpu.VMEM((GCHUNK,), jnp.float32),
            pltpu.VMEM((GCHUNK,), jnp.float32),
            pltpu.VMEM((GCHUNK,), jnp.float32),
            pltpu.SemaphoreType.DMA,
        ],
    )(nb_flat, features, coords_flat)


TN = 200


def _lrelu(x):
    return jnp.where(x >= 0, x, 0.1 * x)


def _tc_body(kp_ref, g_ref, gx_ref, gy_ref, gz_ref, pts_ref,
             wflat_ref, wun_ref, bun_ref, wmlp_ref, bmlp_ref,
             wheads_ref, bheads_ref, f_ref, heads_ref):
    feats = g_ref[...]
    ctr = pts_ref[...]
    dx = gx_ref[...] - ctr[:, 0:1]
    dy = gy_ref[...] - ctr[:, 1:2]
    dz = gz_ref[...] - ctr[:, 2:3]

    x = None
    for p in range(KP):
        ex = dx - kp_ref[p, 0]
        ey = dy - kp_ref[p, 1]
        ez = dz - kp_ref[p, 2]
        sq = ex * ex + ey * ey + ez * ez
        wp = jnp.maximum(1.0 - jnp.sqrt(sq + 1e-9) / KP_EXTENT, 0.0)
        wf = jnp.sum(wp[:, :, None] * feats, axis=1)
        contrib = jnp.dot(wf, wflat_ref[p, :, :],
                          preferred_element_type=jnp.float32)
        x = contrib if x is None else x + contrib
    x = _lrelu(x)
    x = _lrelu(jnp.dot(x, wun_ref[...],
                       preferred_element_type=jnp.float32) + bun_ref[...])
    f = _lrelu(jnp.dot(x, wmlp_ref[...],
                       preferred_element_type=jnp.float32) + bmlp_ref[...])
    f_ref[...] = f
    h = jnp.dot(f, wheads_ref[...],
                preferred_element_type=jnp.float32) + bheads_ref[...]
    col = lax.broadcasted_iota(jnp.int32, h.shape, 1)
    sig = jnp.where(h >= 0, 1.0 / (1.0 + jnp.exp(-h)),
                    jnp.exp(h) / (1.0 + jnp.exp(h)))
    heads_ref[...] = jnp.where(col == 0, sig,
                               jnp.where(col < 1 + OUT_DIM + FREE_DIM,
                                         jnp.maximum(h, 0.0), h))


@jax.jit
def _tc_compute(g3, gx, gy, gz, points, kernel_points, w_kp, w_un, b_un,
                w_mlp, b_mlp, w_heads, b_heads):
    grid = (N // TN,)
    hw = 1 + OUT_DIM + FREE_DIM + C
    return pl.pallas_call(
        _tc_body,
        grid=grid,
        in_specs=[
            pl.BlockSpec(memory_space=pltpu.SMEM),
            pl.BlockSpec((TN, KN, IN_DIM), lambda i: (i, 0, 0)),
            pl.BlockSpec((TN, KN), lambda i: (i, 0)),
            pl.BlockSpec((TN, KN), lambda i: (i, 0)),
            pl.BlockSpec((TN, KN), lambda i: (i, 0)),
            pl.BlockSpec((TN, 3), lambda i: (i, 0)),
            pl.BlockSpec((KP, IN_DIM, OUT_DIM), lambda i: (0, 0, 0)),
            pl.BlockSpec((OUT_DIM, OUT_DIM), lambda i: (0, 0)),
            pl.BlockSpec((1, OUT_DIM), lambda i: (0, 0)),
            pl.BlockSpec((OUT_DIM, FFD), lambda i: (0, 0)),
            pl.BlockSpec((1, FFD), lambda i: (0, 0)),
            pl.BlockSpec((FFD, hw), lambda i: (0, 0)),
            pl.BlockSpec((1, hw), lambda i: (0, 0)),
        ],
        out_specs=[
            pl.BlockSpec((TN, FFD), lambda i: (i, 0)),
            pl.BlockSpec((TN, hw), lambda i: (i, 0)),
        ],
        out_shape=[
            jax.ShapeDtypeStruct((N, FFD), jnp.float32),
            jax.ShapeDtypeStruct((N, hw), jnp.float32),
        ],
    )(kernel_points, g3, gx, gy, gz, points, w_kp, w_un, b_un,
      w_mlp, b_mlp, w_heads, b_heads)


def kernel(points, features, neighbors, kernel_points, W_kp, W_unary, b_unary,
           W_mlp, b_mlp, W_center, b_center, W_var, b_var, W_soft, b_soft):
    coords_flat = jnp.pad(points, ((0, 0), (0, 1))).reshape(-1)
    nb_flat = neighbors.reshape(-1).astype(jnp.int32)
    gf, gx, gy, gz = _sc_gather(nb_flat, features, coords_flat)
    g3 = gf.reshape(N, KN, IN_DIM)
    gx2 = gx.reshape(N, KN)
    gy2 = gy.reshape(N, KN)
    gz2 = gz.reshape(N, KN)

    w_heads = jnp.concatenate([W_center, W_var, W_soft], axis=1)
    b_heads = jnp.concatenate([b_center, b_var, b_soft])[None, :]
    f, heads = _tc_compute(g3, gx2, gy2, gz2, points, kernel_points, W_kp,
                           W_unary, b_unary[None, :], W_mlp, b_mlp[None, :],
                           w_heads, b_heads)
    c = heads[:, 0:1]
    v = heads[:, 1:1 + OUT_DIM + FREE_DIM]
    logits = heads[:, 1 + OUT_DIM + FREE_DIM:]
    return (logits, c, v, f)


# speedup vs baseline: 4.6160x; 2.4532x over previous
import jax
import jax.numpy as jnp
from jax import lax
from jax.experimental import pallas as pl
from jax.experimental.pallas import tpu as pltpu
from jax.experimental.pallas import tpu_sc as plsc

N = 10000
KN = 32
KP = 15
IN_DIM = 128
OUT_DIM = 128
FFD = 128
FREE_DIM = 4
C = 19
KP_EXTENT = 1.2

SC_NC = 2
SC_NS = 16
NW = SC_NC * SC_NS
E = N * KN
PER_W = E // NW
GCHUNK = 400
GITERS = PER_W // GCHUNK


def _sc_gather_body(nb_hbm, feat_hbm, coords_hbm,
                    gf_hbm, gx_hbm, gy_hbm, gz_hbm,
                    coords_v, idx_v, rows_v, gx_v, gy_v, gz_v, sem):
    c = lax.axis_index("c")
    s = lax.axis_index("s")
    wid = s * SC_NC + c
    base = pl.multiple_of(wid * PER_W, 8)
    pltpu.sync_copy(coords_hbm, coords_v)

    def body(j, carry):
        off = pl.multiple_of(base + j * GCHUNK, 8)
        pltpu.sync_copy(nb_hbm.at[pl.ds(off, GCHUNK)], idx_v)
        cp = pltpu.async_copy(feat_hbm.at[idx_v], rows_v, sem)
        for g in range(GCHUNK // 16):
            i16 = idx_v[pl.ds(g * 16, 16)]
            b4 = i16 * 4
            gx_v[pl.ds(g * 16, 16)] = plsc.load_gather(coords_v, [b4])
            gy_v[pl.ds(g * 16, 16)] = plsc.load_gather(coords_v, [b4 + 1])
            gz_v[pl.ds(g * 16, 16)] = plsc.load_gather(coords_v, [b4 + 2])
        cp.wait()
        pltpu.sync_copy(rows_v, gf_hbm.at[pl.ds(off, GCHUNK)])
        pltpu.sync_copy(gx_v, gx_hbm.at[pl.ds(off, GCHUNK)])
        pltpu.sync_copy(gy_v, gy_hbm.at[pl.ds(off, GCHUNK)])
        pltpu.sync_copy(gz_v, gz_hbm.at[pl.ds(off, GCHUNK)])
        return carry

    lax.fori_loop(0, GITERS, body, 0)


@jax.jit
def _sc_gather(nb_flat, features, coords_flat):
    mesh = plsc.VectorSubcoreMesh(core_axis_name="c", subcore_axis_name="s")
    return pl.kernel(
        _sc_gather_body,
        out_type=[
            jax.ShapeDtypeStruct((E, IN_DIM), jnp.float32),
            jax.ShapeDtypeStruct((E,), jnp.float32),
            jax.ShapeDtypeStruct((E,), jnp.float32),
            jax.ShapeDtypeStruct((E,), jnp.float32),
        ],
        mesh=mesh,
        compiler_params=pltpu.CompilerParams(needs_layout_passes=False),
        scratch_types=[
            pltpu.VMEM((4 * N,), jnp.float32),
            pltpu.VMEM((GCHUNK,), jnp.int32),
            pltpu.VMEM((GCHUNK, IN_DIM), jnp.float32),
            pltpu.VMEM((GCHUNK,), jnp.float32),
            pltpu.VMEM((GCHUNK,), jnp.float32),
            pltpu.VMEM((GCHUNK,), jnp.float32),
            pltpu.SemaphoreType.DMA,
        ],
    )(nb_flat, features, coords_flat)


TN = 200


def _lrelu(x):
    return jnp.where(x >= 0, x, 0.1 * x)


TB = 8


def _tc_body(kp_ref, g_ref, gx_ref, gy_ref, gz_ref, pts_ref,
             wflat_ref, wun_ref, bun_ref, wmlp_ref, bmlp_ref,
             wheads_ref, bheads_ref, f_ref, heads_ref):
    feats = g_ref[...]
    ctr = pts_ref[...]
    dx = gx_ref[...] - ctr[:, 0:1]
    dy = gy_ref[...] - ctr[:, 1:2]
    dz = gz_ref[...] - ctr[:, 2:3]

    LW = TB * KN
    sub_i = lax.broadcasted_iota(jnp.int32, (TN, LW), 0) % TB
    lane_i = lax.broadcasted_iota(jnp.int32, (TN, LW), 1) // KN
    maskf = jnp.where(sub_i == lane_i, 1.0, 0.0)

    wlist = []
    for p in range(KP):
        ex = dx - kp_ref[p, 0]
        ey = dy - kp_ref[p, 1]
        ez = dz - kp_ref[p, 2]
        sq = ex * ex + ey * ey + ez * ez
        wp = jnp.maximum(1.0 - jnp.sqrt(sq + 1e-9) / KP_EXTENT, 0.0)
        wlist.append(jnp.tile(wp, (1, TB)) * maskf)
    at_all = jnp.stack(wlist, axis=0)

    NB = TN // TB
    hblocks = []
    for b in range(NB):
        atb = at_all[:, b * TB:(b + 1) * TB, :].reshape(KP * TB, LW)
        gb = feats[b * TB:(b + 1) * TB].reshape(LW, IN_DIM)
        hblocks.append(jnp.dot(atb, gb, preferred_element_type=jnp.float32))
    h_all = jnp.stack(hblocks, axis=0)

    x = None
    for p in range(KP):
        hp = h_all[:, p * TB:(p + 1) * TB, :].reshape(TN, IN_DIM)
        contrib = jnp.dot(hp, wflat_ref[p, :, :],
                          preferred_element_type=jnp.float32)
        x = contrib if x is None else x + contrib
    x = _lrelu(x)
    x = _lrelu(jnp.dot(x, wun_ref[...],
                       preferred_element_type=jnp.float32) + bun_ref[...])
    f = _lrelu(jnp.dot(x, wmlp_ref[...],
                       preferred_element_type=jnp.float32) + bmlp_ref[...])
    f_ref[...] = f
    h = jnp.dot(f, wheads_ref[...],
                preferred_element_type=jnp.float32) + bheads_ref[...]
    col = lax.broadcasted_iota(jnp.int32, h.shape, 1)
    sig = jnp.where(h >= 0, 1.0 / (1.0 + jnp.exp(-h)),
                    jnp.exp(h) / (1.0 + jnp.exp(h)))
    heads_ref[...] = jnp.where(col == 0, sig,
                               jnp.where(col < 1 + OUT_DIM + FREE_DIM,
                                         jnp.maximum(h, 0.0), h))


@jax.jit
def _tc_compute(g3, gx, gy, gz, points, kernel_points, w_kp, w_un, b_un,
                w_mlp, b_mlp, w_heads, b_heads):
    grid = (N // TN,)
    hw = 1 + OUT_DIM + FREE_DIM + C
    return pl.pallas_call(
        _tc_body,
        grid=grid,
        in_specs=[
            pl.BlockSpec(memory_space=pltpu.SMEM),
            pl.BlockSpec((TN, KN, IN_DIM), lambda i: (i, 0, 0)),
            pl.BlockSpec((TN, KN), lambda i: (i, 0)),
            pl.BlockSpec((TN, KN), lambda i: (i, 0)),
            pl.BlockSpec((TN, KN), lambda i: (i, 0)),
            pl.BlockSpec((TN, 3), lambda i: (i, 0)),
            pl.BlockSpec((KP, IN_DIM, OUT_DIM), lambda i: (0, 0, 0)),
            pl.BlockSpec((OUT_DIM, OUT_DIM), lambda i: (0, 0)),
            pl.BlockSpec((1, OUT_DIM), lambda i: (0, 0)),
            pl.BlockSpec((OUT_DIM, FFD), lambda i: (0, 0)),
            pl.BlockSpec((1, FFD), lambda i: (0, 0)),
            pl.BlockSpec((FFD, hw), lambda i: (0, 0)),
            pl.BlockSpec((1, hw), lambda i: (0, 0)),
        ],
        out_specs=[
            pl.BlockSpec((TN, FFD), lambda i: (i, 0)),
            pl.BlockSpec((TN, hw), lambda i: (i, 0)),
        ],
        out_shape=[
            jax.ShapeDtypeStruct((N, FFD), jnp.float32),
            jax.ShapeDtypeStruct((N, hw), jnp.float32),
        ],
    )(kernel_points, g3, gx, gy, gz, points, w_kp, w_un, b_un,
      w_mlp, b_mlp, w_heads, b_heads)


def kernel(points, features, neighbors, kernel_points, W_kp, W_unary, b_unary,
           W_mlp, b_mlp, W_center, b_center, W_var, b_var, W_soft, b_soft):
    coords_flat = jnp.pad(points, ((0, 0), (0, 1))).reshape(-1)
    nb_flat = neighbors.reshape(-1).astype(jnp.int32)
    gf, gx, gy, gz = _sc_gather(nb_flat, features, coords_flat)
    g3 = gf.reshape(N, KN, IN_DIM)
    gx2 = gx.reshape(N, KN)
    gy2 = gy.reshape(N, KN)
    gz2 = gz.reshape(N, KN)

    w_heads = jnp.concatenate([W_center, W_var, W_soft], axis=1)
    b_heads = jnp.concatenate([b_center, b_var, b_soft])[None, :]
    f, heads = _tc_compute(g3, gx2, gy2, gz2, points, kernel_points, W_kp,
                           W_unary, b_unary[None, :], W_mlp, b_mlp[None, :],
                           w_heads, b_heads)
    c = heads[:, 0:1]
    v = heads[:, 1:1 + OUT_DIM + FREE_DIM]
    logits = heads[:, 1 + OUT_DIM + FREE_DIM:]
    return (logits, c, v, f)


# speedup vs baseline: 5.5639x; 1.2053x over previous
import jax
import jax.numpy as jnp
from jax import lax
from jax.experimental import pallas as pl
from jax.experimental.pallas import tpu as pltpu
from jax.experimental.pallas import tpu_sc as plsc

N = 10000
KN = 32
KP = 15
IN_DIM = 128
OUT_DIM = 128
FFD = 128
FREE_DIM = 4
C = 19
KP_EXTENT = 1.2

SC_NC = 2
SC_NS = 16
NW = SC_NC * SC_NS
E = N * KN
PER_W = E // NW
GCHUNK = 400
GITERS = PER_W // GCHUNK


def _sc_gather_body(nb_hbm, table_hbm, gf_hbm, gx_hbm, gy_hbm, gz_hbm,
                    idx_all, rows0, rows1, gx_v, gy_v, gz_v,
                    gx_w, gy_w, gz_w, sem0, sem1, semw):
    c = lax.axis_index("c")
    s = lax.axis_index("s")
    wid = s * SC_NC + c
    base = pl.multiple_of(wid * PER_W, 8)
    pltpu.sync_copy(nb_hbm.at[pl.ds(base, PER_W)], idx_all)

    def start(j, rows, sem):
        off = pl.multiple_of(j * GCHUNK, 8)
        pltpu.async_copy(table_hbm.at[idx_all.at[pl.ds(off, GCHUNK)]],
                         rows, sem)

    def wait(rows, sem):
        pltpu.make_async_copy(table_hbm.at[pl.ds(0, GCHUNK)], rows,
                              sem).wait()

    iot = lax.iota(jnp.int32, 16)
    c64 = iot * 0 + (2 * (IN_DIM // 4))

    def drain(j, rows, gxv, gyv, gzv, sem):
        wait(rows, sem)
        for g in range(GCHUNK // 16):
            r16 = iot + g * 16
            xi = plsc.load_gather(rows, [r16, c64])
            yi = plsc.load_gather(rows, [r16, c64 + 1])
            zi = plsc.load_gather(rows, [r16, c64 + 2])
            gxv[pl.ds(g * 16, 16)] = plsc.bitcast(xi, jnp.float32)
            gyv[pl.ds(g * 16, 16)] = plsc.bitcast(yi, jnp.float32)
            gzv[pl.ds(g * 16, 16)] = plsc.bitcast(zi, jnp.float32)
        off = pl.multiple_of(base + j * GCHUNK, 8)
        return (pltpu.async_copy(rows, gf_hbm.at[pl.ds(off, GCHUNK)], semw),
                pltpu.async_copy(gxv, gx_hbm.at[pl.ds(off, GCHUNK)], semw),
                pltpu.async_copy(gyv, gy_hbm.at[pl.ds(off, GCHUNK)], semw),
                pltpu.async_copy(gzv, gz_hbm.at[pl.ds(off, GCHUNK)], semw))

    start(0, rows0, sem0)

    def body(j2, carry):
        j = 2 * j2
        start(j + 1, rows1, sem1)
        ws0 = drain(j, rows0, gx_v, gy_v, gz_v, sem0)
        for h in ws0:
            h.wait()
        start(j + 2, rows0, sem0)
        ws1 = drain(j + 1, rows1, gx_w, gy_w, gz_w, sem1)
        for h in ws1:
            h.wait()
        return carry

    lax.fori_loop(0, GITERS // 2, body, 0)
    ws = drain(GITERS - 1, rows0, gx_v, gy_v, gz_v, sem0)
    for h in ws:
        h.wait()


@jax.jit
def _sc_gather(nb_flat, table):
    mesh = plsc.VectorSubcoreMesh(core_axis_name="c", subcore_axis_name="s")
    return pl.kernel(
        _sc_gather_body,
        out_type=[
            jax.ShapeDtypeStruct((E, IN_DIM), jnp.int32),
            jax.ShapeDtypeStruct((E,), jnp.float32),
            jax.ShapeDtypeStruct((E,), jnp.float32),
            jax.ShapeDtypeStruct((E,), jnp.float32),
        ],
        mesh=mesh,
        compiler_params=pltpu.CompilerParams(needs_layout_passes=False),
        scratch_types=[
            pltpu.VMEM((PER_W,), jnp.int32),
            pltpu.VMEM((GCHUNK, IN_DIM), jnp.int32),
            pltpu.VMEM((GCHUNK, IN_DIM), jnp.int32),
            pltpu.VMEM((GCHUNK,), jnp.float32),
            pltpu.VMEM((GCHUNK,), jnp.float32),
            pltpu.VMEM((GCHUNK,), jnp.float32),
            pltpu.VMEM((GCHUNK,), jnp.float32),
            pltpu.VMEM((GCHUNK,), jnp.float32),
            pltpu.VMEM((GCHUNK,), jnp.float32),
            pltpu.SemaphoreType.DMA,
            pltpu.SemaphoreType.DMA,
            pltpu.SemaphoreType.DMA,
        ],
    )(nb_flat, table)


TN = 512
TB = 8
LW = TB * KN
NBLK = TN // TB


def _lrelu(x):
    return jnp.where(x >= 0, x, 0.1 * x)


def _tc_body(kp_ref, g_ref, gx_ref, gy_ref, gz_ref, pts_ref,
             wflat_ref, wun_ref, bun_ref, wmlp_ref, bmlp_ref,
             wheads_ref, bheads_ref, f_ref, heads_ref):
    gi = g_ref[...][:, :IN_DIM // 2]
    lo_f = lax.bitcast_convert_type(lax.shift_left(gi, 16), jnp.float32)
    hi_f = lax.bitcast_convert_type(
        jnp.bitwise_and(gi, jnp.int32(-65536)), jnp.float32)
    feats = jnp.concatenate([lo_f, hi_f], axis=1)
    pts = pts_ref[...]

    pxr = jnp.repeat(pts[:, 0:1].reshape(NBLK, TB), KN, axis=1)
    pyr = jnp.repeat(pts[:, 1:2].reshape(NBLK, TB), KN, axis=1)
    pzr = jnp.repeat(pts[:, 2:3].reshape(NBLK, TB), KN, axis=1)
    dx = gx_ref[...] - pxr
    dy = gy_ref[...] - pyr
    dz = gz_ref[...] - pzr

    sub_i = lax.broadcasted_iota(jnp.int32, (TB, LW), 0)
    lane_i = lax.broadcasted_iota(jnp.int32, (TB, LW), 1) // KN
    mask8 = jnp.where(sub_i == lane_i, 1.0, 0.0)

    atl = []
    for p in range(KP):
        ex = dx - kp_ref[p, 0]
        ey = dy - kp_ref[p, 1]
        ez = dz - kp_ref[p, 2]
        sq = ex * ex + ey * ey + ez * ez
        wp = jnp.maximum(1.0 - jnp.sqrt(sq + 1e-9) / KP_EXTENT, 0.0)
        atl.append(wp[:, None, :] * mask8[None, :, :])
    at_all = jnp.stack(atl, axis=0)

    hblocks = []
    for b in range(NBLK):
        atb = at_all[:, b].reshape(KP * TB, LW)
        gb = feats[b * LW:(b + 1) * LW, :]
        hblocks.append(jnp.dot(atb, gb, preferred_element_type=jnp.float32))
    h_all = jnp.stack(hblocks, axis=0)

    x = None
    for p in range(KP):
        hp = h_all[:, p * TB:(p + 1) * TB, :].reshape(TN, IN_DIM)
        contrib = jnp.dot(hp, wflat_ref[p, :, :],
                          preferred_element_type=jnp.float32)
        x = contrib if x is None else x + contrib
    x = _lrelu(x)
    x = _lrelu(jnp.dot(x, wun_ref[...],
                       preferred_element_type=jnp.float32) + bun_ref[...])
    f = _lrelu(jnp.dot(x, wmlp_ref[...],
                       preferred_element_type=jnp.float32) + bmlp_ref[...])
    f_ref[...] = f
    h = jnp.dot(f, wheads_ref[...],
                preferred_element_type=jnp.float32) + bheads_ref[...]
    col = lax.broadcasted_iota(jnp.int32, h.shape, 1)
    sig = jnp.where(h >= 0, 1.0 / (1.0 + jnp.exp(-h)),
                    jnp.exp(h) / (1.0 + jnp.exp(h)))
    heads_ref[...] = jnp.where(col == 0, sig,
                               jnp.where(col < 1 + OUT_DIM + FREE_DIM,
                                         jnp.maximum(h, 0.0), h))


@jax.jit
def _tc_compute(g3, gx, gy, gz, points, kernel_points, w_kp, w_un, b_un,
                w_mlp, b_mlp, w_heads, b_heads):
    grid = (pl.cdiv(N, TN),)
    hw = 1 + OUT_DIM + FREE_DIM + C
    return pl.pallas_call(
        _tc_body,
        grid=grid,
        in_specs=[
            pl.BlockSpec(memory_space=pltpu.SMEM),
            pl.BlockSpec((TN * KN, IN_DIM), lambda i: (i, 0)),
            pl.BlockSpec((NBLK, LW), lambda i: (i, 0)),
            pl.BlockSpec((NBLK, LW), lambda i: (i, 0)),
            pl.BlockSpec((NBLK, LW), lambda i: (i, 0)),
            pl.BlockSpec((TN, 3), lambda i: (i, 0)),
            pl.BlockSpec((KP, IN_DIM, OUT_DIM), lambda i: (0, 0, 0)),
            pl.BlockSpec((OUT_DIM, OUT_DIM), lambda i: (0, 0)),
            pl.BlockSpec((1, OUT_DIM), lambda i: (0, 0)),
            pl.BlockSpec((OUT_DIM, FFD), lambda i: (0, 0)),
            pl.BlockSpec((1, FFD), lambda i: (0, 0)),
            pl.BlockSpec((FFD, hw), lambda i: (0, 0)),
            pl.BlockSpec((1, hw), lambda i: (0, 0)),
        ],
        out_specs=[
            pl.BlockSpec((TN, FFD), lambda i: (i, 0)),
            pl.BlockSpec((TN, hw), lambda i: (i, 0)),
        ],
        out_shape=[
            jax.ShapeDtypeStruct((N, FFD), jnp.float32),
            jax.ShapeDtypeStruct((N, hw), jnp.float32),
        ],
    )(kernel_points, g3, gx, gy, gz, points, w_kp, w_un, b_un,
      w_mlp, b_mlp, w_heads, b_heads)


def kernel(points, features, neighbors, kernel_points, W_kp, W_unary, b_unary,
           W_mlp, b_mlp, W_center, b_center, W_var, b_var, W_soft, b_soft):
    fb = features.astype(jnp.bfloat16)
    fi = lax.bitcast_convert_type(
        jnp.stack([fb[:, :IN_DIM // 2], fb[:, IN_DIM // 2:]], axis=-1),
        jnp.int32)
    ci = lax.bitcast_convert_type(points, jnp.int32)
    table = jnp.concatenate(
        [fi, ci, jnp.zeros((N, IN_DIM - IN_DIM // 2 - 3), jnp.int32)],
        axis=1)
    nb_flat = neighbors.reshape(-1).astype(jnp.int32)
    gf, gx, gy, gz = _sc_gather(nb_flat, table)
    g3 = gf
    gx2 = gx.reshape(E // LW, LW)
    gy2 = gy.reshape(E // LW, LW)
    gz2 = gz.reshape(E // LW, LW)

    w_heads = jnp.concatenate([W_center, W_var, W_soft], axis=1)
    b_heads = jnp.concatenate([b_center, b_var, b_soft])[None, :]
    f, heads = _tc_compute(g3, gx2, gy2, gz2, points, kernel_points, W_kp,
                           W_unary, b_unary[None, :], W_mlp, b_mlp[None, :],
                           w_heads, b_heads)
    c = heads[:, 0:1]
    v = heads[:, 1:1 + OUT_DIM + FREE_DIM]
    logits = heads[:, 1 + OUT_DIM + FREE_DIM:]
    return (logits, c, v, f)


# speedup vs baseline: 5.7702x; 1.0371x over previous
import jax
import jax.numpy as jnp
from jax import lax
from jax.experimental import pallas as pl
from jax.experimental.pallas import tpu as pltpu
from jax.experimental.pallas import tpu_sc as plsc

N = 10000
KN = 32
KP = 15
IN_DIM = 128
OUT_DIM = 128
FFD = 128
FREE_DIM = 4
C = 19
KP_EXTENT = 1.2

SC_NC = 2
SC_NS = 16
NW = SC_NC * SC_NS
E = N * KN
PER_W = E // NW
GCHUNK = 400
GITERS = PER_W // GCHUNK


def _sc_gather_body(nb_hbm, table_hbm, gf_hbm, gx_hbm, gy_hbm, gz_hbm,
                    idx_all, rows0, rows1, gx_v, gy_v, gz_v,
                    gx_w, gy_w, gz_w, sem0, sem1, semw):
    c = lax.axis_index("c")
    s = lax.axis_index("s")
    wid = s * SC_NC + c
    base = pl.multiple_of(wid * PER_W, 8)
    pltpu.sync_copy(nb_hbm.at[pl.ds(base, PER_W)], idx_all)

    def start(j, rows, sem):
        off = pl.multiple_of(j * GCHUNK, 8)
        pltpu.async_copy(table_hbm.at[idx_all.at[pl.ds(off, GCHUNK)]],
                         rows, sem)

    def wait(rows, sem):
        pltpu.make_async_copy(table_hbm.at[pl.ds(0, GCHUNK)], rows,
                              sem).wait()

    iot = lax.iota(jnp.int32, 16)
    c64 = iot * 0 + (2 * (IN_DIM // 4))

    def drain(j, rows, gxv, gyv, gzv, sem):
        wait(rows, sem)
        for g in range(GCHUNK // 16):
            r16 = iot + g * 16
            xi = plsc.load_gather(rows, [r16, c64])
            yi = plsc.load_gather(rows, [r16, c64 + 1])
            zi = plsc.load_gather(rows, [r16, c64 + 2])
            gxv[pl.ds(g * 16, 16)] = plsc.bitcast(xi, jnp.float32)
            gyv[pl.ds(g * 16, 16)] = plsc.bitcast(yi, jnp.float32)
            gzv[pl.ds(g * 16, 16)] = plsc.bitcast(zi, jnp.float32)
        off = pl.multiple_of(base + j * GCHUNK, 8)
        return (pltpu.async_copy(rows, gf_hbm.at[pl.ds(off, GCHUNK)], semw),
                pltpu.async_copy(gxv, gx_hbm.at[pl.ds(off, GCHUNK)], semw),
                pltpu.async_copy(gyv, gy_hbm.at[pl.ds(off, GCHUNK)], semw),
                pltpu.async_copy(gzv, gz_hbm.at[pl.ds(off, GCHUNK)], semw))

    start(0, rows0, sem0)

    def body(j2, carry):
        j = 2 * j2
        start(j + 1, rows1, sem1)
        ws0 = drain(j, rows0, gx_v, gy_v, gz_v, sem0)
        for h in ws0:
            h.wait()
        start(j + 2, rows0, sem0)
        ws1 = drain(j + 1, rows1, gx_w, gy_w, gz_w, sem1)
        for h in ws1:
            h.wait()
        return carry

    lax.fori_loop(0, GITERS // 2, body, 0)
    ws = drain(GITERS - 1, rows0, gx_v, gy_v, gz_v, sem0)
    for h in ws:
        h.wait()


@jax.jit
def _sc_gather(nb_flat, table):
    mesh = plsc.VectorSubcoreMesh(core_axis_name="c", subcore_axis_name="s")
    return pl.kernel(
        _sc_gather_body,
        out_type=[
            jax.ShapeDtypeStruct((E, IN_DIM), jnp.int32),
            jax.ShapeDtypeStruct((E,), jnp.float32),
            jax.ShapeDtypeStruct((E,), jnp.float32),
            jax.ShapeDtypeStruct((E,), jnp.float32),
        ],
        mesh=mesh,
        compiler_params=pltpu.CompilerParams(needs_layout_passes=False),
        scratch_types=[
            pltpu.VMEM((PER_W,), jnp.int32),
            pltpu.VMEM((GCHUNK, IN_DIM), jnp.int32),
            pltpu.VMEM((GCHUNK, IN_DIM), jnp.int32),
            pltpu.VMEM((GCHUNK,), jnp.float32),
            pltpu.VMEM((GCHUNK,), jnp.float32),
            pltpu.VMEM((GCHUNK,), jnp.float32),
            pltpu.VMEM((GCHUNK,), jnp.float32),
            pltpu.VMEM((GCHUNK,), jnp.float32),
            pltpu.VMEM((GCHUNK,), jnp.float32),
            pltpu.SemaphoreType.DMA,
            pltpu.SemaphoreType.DMA,
            pltpu.SemaphoreType.DMA,
        ],
    )(nb_flat, table)


TN = 512
TB = 8
LW = TB * KN
NBLK = TN // TB


def _lrelu(x):
    return jnp.where(x >= 0, x, 0.1 * x)


def _tc_body(kp_ref, g_ref, gx_ref, gy_ref, gz_ref, pts_ref,
             wflat_ref, wun_ref, bun_ref, wmlp_ref, bmlp_ref,
             wheads_ref, bheads_ref, f_ref, v_ref, log_ref, c_ref):
    gi = g_ref[...][:, :IN_DIM // 2]
    lo_f = lax.bitcast_convert_type(lax.shift_left(gi, 16), jnp.float32)
    hi_f = lax.bitcast_convert_type(
        jnp.bitwise_and(gi, jnp.int32(-65536)), jnp.float32)
    feats = jnp.concatenate([lo_f, hi_f], axis=1)
    pts = pts_ref[...]

    pxr = jnp.repeat(pts[:, 0:1].reshape(NBLK, TB), KN, axis=1)
    pyr = jnp.repeat(pts[:, 1:2].reshape(NBLK, TB), KN, axis=1)
    pzr = jnp.repeat(pts[:, 2:3].reshape(NBLK, TB), KN, axis=1)
    dx = gx_ref[...] - pxr
    dy = gy_ref[...] - pyr
    dz = gz_ref[...] - pzr

    sub_i = lax.broadcasted_iota(jnp.int32, (TB, LW), 0)
    lane_i = lax.broadcasted_iota(jnp.int32, (TB, LW), 1) // KN
    mask8 = jnp.where(sub_i == lane_i, 1.0, 0.0)

    atl = []
    for p in range(KP):
        ex = dx - kp_ref[p, 0]
        ey = dy - kp_ref[p, 1]
        ez = dz - kp_ref[p, 2]
        sq = ex * ex + ey * ey + ez * ez
        wp = jnp.maximum(1.0 - jnp.sqrt(sq + 1e-9) / KP_EXTENT, 0.0)
        atl.append(wp[:, None, :] * mask8[None, :, :])
    at_all = jnp.stack(atl, axis=0)

    hblocks = []
    for b in range(NBLK):
        atb = at_all[:, b].reshape(KP * TB, LW)
        gb = feats[b * LW:(b + 1) * LW, :]
        hblocks.append(jnp.dot(atb, gb, preferred_element_type=jnp.float32))
    h_all = jnp.stack(hblocks, axis=0)

    x = None
    for p in range(KP):
        hp = h_all[:, p * TB:(p + 1) * TB, :].reshape(TN, IN_DIM)
        contrib = jnp.dot(hp, wflat_ref[p, :, :],
                          preferred_element_type=jnp.float32)
        x = contrib if x is None else x + contrib
    x = _lrelu(x)
    x = _lrelu(jnp.dot(x, wun_ref[...],
                       preferred_element_type=jnp.float32) + bun_ref[...])
    f = _lrelu(jnp.dot(x, wmlp_ref[...],
                       preferred_element_type=jnp.float32) + bmlp_ref[...])
    f_ref[...] = f
    h = jnp.dot(f, wheads_ref[...],
                preferred_element_type=jnp.float32) + bheads_ref[...]
    VW = OUT_DIM + FREE_DIM
    v_ref[...] = jnp.maximum(h[:, :VW], 0.0)
    log_ref[...] = h[:, VW:VW + C]
    hc = h[:, VW + C:]
    c_ref[...] = jnp.where(hc >= 0, 1.0 / (1.0 + jnp.exp(-hc)),
                           jnp.exp(hc) / (1.0 + jnp.exp(hc)))


@jax.jit
def _tc_compute(g3, gx, gy, gz, points, kernel_points, w_kp, w_un, b_un,
                w_mlp, b_mlp, w_heads, b_heads):
    grid = (pl.cdiv(N, TN),)
    hw = 1 + OUT_DIM + FREE_DIM + C
    return pl.pallas_call(
        _tc_body,
        grid=grid,
        in_specs=[
            pl.BlockSpec(memory_space=pltpu.SMEM),
            pl.BlockSpec((TN * KN, IN_DIM), lambda i: (i, 0)),
            pl.BlockSpec((NBLK, LW), lambda i: (i, 0)),
            pl.BlockSpec((NBLK, LW), lambda i: (i, 0)),
            pl.BlockSpec((NBLK, LW), lambda i: (i, 0)),
            pl.BlockSpec((TN, 3), lambda i: (i, 0)),
            pl.BlockSpec((KP, IN_DIM, OUT_DIM), lambda i: (0, 0, 0)),
            pl.BlockSpec((OUT_DIM, OUT_DIM), lambda i: (0, 0)),
            pl.BlockSpec((1, OUT_DIM), lambda i: (0, 0)),
            pl.BlockSpec((OUT_DIM, FFD), lambda i: (0, 0)),
            pl.BlockSpec((1, FFD), lambda i: (0, 0)),
            pl.BlockSpec((FFD, hw), lambda i: (0, 0)),
            pl.BlockSpec((1, hw), lambda i: (0, 0)),
        ],
        out_specs=[
            pl.BlockSpec((TN, FFD), lambda i: (i, 0)),
            pl.BlockSpec((TN, OUT_DIM + FREE_DIM), lambda i: (i, 0)),
            pl.BlockSpec((TN, C), lambda i: (i, 0)),
            pl.BlockSpec((TN, 1), lambda i: (i, 0)),
        ],
        out_shape=[
            jax.ShapeDtypeStruct((N, FFD), jnp.float32),
            jax.ShapeDtypeStruct((N, OUT_DIM + FREE_DIM), jnp.float32),
            jax.ShapeDtypeStruct((N, C), jnp.float32),
            jax.ShapeDtypeStruct((N, 1), jnp.float32),
        ],
    )(kernel_points, g3, gx, gy, gz, points, w_kp, w_un, b_un,
      w_mlp, b_mlp, w_heads, b_heads)


def kernel(points, features, neighbors, kernel_points, W_kp, W_unary, b_unary,
           W_mlp, b_mlp, W_center, b_center, W_var, b_var, W_soft, b_soft):
    fb = features.astype(jnp.bfloat16)
    fi = lax.bitcast_convert_type(
        jnp.stack([fb[:, :IN_DIM // 2], fb[:, IN_DIM // 2:]], axis=-1),
        jnp.int32)
    ci = lax.bitcast_convert_type(points, jnp.int32)
    table = jnp.concatenate(
        [fi, ci, jnp.zeros((N, IN_DIM - IN_DIM // 2 - 3), jnp.int32)],
        axis=1)
    nb_flat = neighbors.reshape(-1).astype(jnp.int32)
    gf, gx, gy, gz = _sc_gather(nb_flat, table)
    g3 = gf
    gx2 = gx.reshape(E // LW, LW)
    gy2 = gy.reshape(E // LW, LW)
    gz2 = gz.reshape(E // LW, LW)

    w_heads = jnp.concatenate([W_var, W_soft, W_center], axis=1)
    b_heads = jnp.concatenate([b_var, b_soft, b_center])[None, :]
    f, v, logits, c = _tc_compute(g3, gx2, gy2, gz2, points, kernel_points,
                                  W_kp, W_unary, b_unary[None, :], W_mlp,
                                  b_mlp[None, :], w_heads, b_heads)
    return (logits, c, v, f)
